# baseline (speedup 1.0000x reference)
import functools

import jax
import jax.numpy as jnp
from jax import lax
from jax.experimental import pallas as pl
from jax.experimental.pallas import tpu as pltpu
from jax.experimental.pallas import tpu_sc as plsc

NC = 2
NS = 16
NW = NC * NS
XPAD = 128
HPAD = 56
GRP = 8
LANES = 16


@functools.cache
def _build(batch: int, hist: int, dim: int):
    rows_per_w = batch // NW
    n_tiles = HPAD // GRP
    mesh = plsc.VectorSubcoreMesh(core_axis_name="c", subcore_axis_name="s")
    nvec = dim // LANES

    @functools.partial(
        pl.kernel,
        out_type=jax.ShapeDtypeStruct((batch, HPAD, dim), jnp.float32),
        mesh=mesh,
        scratch_types=[
            pltpu.VMEM((rows_per_w, XPAD), jnp.int32),
            pltpu.VMEM((n_tiles, GRP, GRP, dim), jnp.float32),
            pltpu.VMEM((n_tiles, GRP, dim), jnp.float32),
            [pltpu.SemaphoreType.DMA] * n_tiles,
            [pltpu.SemaphoreType.DMA] * n_tiles,
        ],
        compiler_params=pltpu.CompilerParams(use_tc_tiling_on_sc=True),
    )
    def emb_kernel(x_hbm, w_hbm, out_hbm, idx_v, grp_v, stag_v,
                   gsems, osems):
        wid = lax.axis_index("s") * NC + lax.axis_index("c")
        base = wid * rows_per_w
        pltpu.sync_copy(x_hbm.at[pl.ds(base, rows_per_w), :], idx_v)

        def start_gather(k, t):
            iv = idx_v[k, pl.ds((t // 2) * LANES, LANES)]
            for s in range(GRP):
                g = jax.lax.shift_right_logical(iv[(t % 2) * GRP + s], 3)
                pltpu.async_copy(w_hbm.at[g], grp_v.at[t, s], gsems[t])

        for t in range(n_tiles):
            start_gather(0, t)

        def row_step(k, carry):
            for t in range(n_tiles):
                for s in range(GRP):
                    pltpu.make_async_copy(
                        w_hbm.at[0], grp_v.at[t, s], gsems[t]
                    ).wait()

                @pl.when(k > 0)
                def _():
                    pltpu.make_async_copy(
                        stag_v.at[t],
                        out_hbm.at[base + k - 1, pl.ds(t * GRP, GRP), :],
                        osems[t],
                    ).wait()

                iv = idx_v[k, pl.ds((t // 2) * LANES, LANES)]
                for s in range(GRP):
                    src = iv[(t % 2) * GRP + s] & (GRP - 1)
                    for c in range(nvec):
                        sl = pl.ds(c * LANES, LANES)
                        stag_v[t, s, sl] = grp_v[t, s, src, sl]

                pltpu.async_copy(
                    stag_v.at[t],
                    out_hbm.at[base + k, pl.ds(t * GRP, GRP), :],
                    osems[t],
                )

                @pl.when(k + 1 < rows_per_w)
                def _():
                    start_gather(k + 1, t)

            return carry

        lax.fori_loop(0, rows_per_w, row_step, 0, unroll=False)

        for t in range(n_tiles):
            pltpu.make_async_copy(
                stag_v.at[t],
                out_hbm.at[base + rows_per_w - 1, pl.ds(t * GRP, GRP), :],
                osems[t],
            ).wait()

    return emb_kernel


def kernel(x, weight):
    batch, hist = x.shape
    n_rows, dim = weight.shape
    xpad = jnp.pad(x.astype(jnp.int32), ((0, 0), (0, XPAD - hist)))
    w3 = weight.reshape(n_rows // GRP, GRP, dim)
    out = _build(batch, hist, dim)(xpad, w3)
    return out[:, :hist, :]


# speedup vs baseline: 1.9542x; 1.9542x over previous
import functools

import jax
import jax.numpy as jnp
from jax import lax
from jax.experimental import pallas as pl
from jax.experimental.pallas import tpu as pltpu
from jax.experimental.pallas import tpu_sc as plsc

NC = 2
NS = 16
NW = NC * NS
CHUNK = 128
PADDIM = 128
LANES = 16
NBUF = 5
LEAD = 2
PR = 128
PB = 2


@functools.cache
def _build_detile(n_rows: int, dim: int):
    rows_w = -(-n_rows // NW)
    rows_w = -(-rows_w // 8) * 8
    n_blk = -(-rows_w // PR)
    n_blk = -(-n_blk // PB) * PB
    nvec = dim // LANES
    mesh = plsc.VectorSubcoreMesh(core_axis_name="c", subcore_axis_name="s")

    @functools.partial(
        pl.kernel,
        out_type=jax.ShapeDtypeStruct((n_rows, PADDIM), jnp.float32),
        mesh=mesh,
        scratch_types=[
            pltpu.VMEM((PB, PR, dim), jnp.float32),
            pltpu.VMEM((PB, PR, PADDIM), jnp.float32),
            [pltpu.SemaphoreType.DMA] * PB,
            [pltpu.SemaphoreType.DMA] * PB,
        ],
        compiler_params=pltpu.CompilerParams(use_tc_tiling_on_sc=True),
    )
    def detile_kernel(w_hbm, out_hbm, in_v, out_v, isems, osems):
        wid = lax.axis_index("s") * NC + lax.axis_index("c")
        start = wid * rows_w
        end = jnp.minimum(start + rows_w, n_rows)

        def blk_start(c):
            return jnp.minimum(start + c * PR, end - PR)

        def fire_in(c, p):
            pltpu.async_copy(
                w_hbm.at[pl.ds(blk_start(c), PR), :], in_v.at[p], isems[p]
            )

        for p in range(PB):
            fire_in(p, p)

        def step(g, carry):
            for p in range(PB):
                c = g * PB + p
                pltpu.make_async_copy(
                    w_hbm.at[pl.ds(0, PR), :], in_v.at[p], isems[p]
                ).wait()

                @pl.when(c >= PB)
                def _():
                    pltpu.make_async_copy(
                        out_v.at[p],
                        out_hbm.at[pl.ds(0, PR), :],
                        osems[p],
                    ).wait()

                def row(i, carry2):
                    for v in range(nvec):
                        sl = pl.ds(v * LANES, LANES)
                        out_v[p, i, sl] = in_v[p, i, sl]
                    return carry2

                lax.fori_loop(0, PR, row, 0, unroll=False)
                pltpu.async_copy(
                    out_v.at[p],
                    out_hbm.at[pl.ds(blk_start(c), PR), :],
                    osems[p],
                )

                @pl.when(c + PB < n_blk)
                def _():
                    fire_in(c + PB, p)

            return carry

        lax.fori_loop(0, n_blk // PB, step, 0, unroll=False)

        for p in range(PB):
            pltpu.make_async_copy(
                out_v.at[p], out_hbm.at[pl.ds(0, PR), :], osems[p]
            ).wait()

    return detile_kernel


@functools.cache
def _build_gather(n_chunks: int, dim: int):
    assert n_chunks % NBUF == 0
    mesh = plsc.VectorSubcoreMesh(core_axis_name="c", subcore_axis_name="s")

    @functools.partial(
        pl.kernel,
        out_type=jax.ShapeDtypeStruct((NW, n_chunks, CHUNK, dim), jnp.float32),
        mesh=mesh,
        scratch_types=[
            pltpu.VMEM((n_chunks, CHUNK), jnp.int32),
            pltpu.VMEM((NBUF, CHUNK, PADDIM), jnp.float32),
            [pltpu.SemaphoreType.DMA] * NBUF,
            [pltpu.SemaphoreType.DMA] * NBUF,
        ],
        compiler_params=pltpu.CompilerParams(use_tc_tiling_on_sc=False),
    )
    def emb_kernel(x_hbm, w_hbm, out_hbm, idx_v, rows_v, gsems, osems):
        wid = lax.axis_index("s") * NC + lax.axis_index("c")
        pltpu.sync_copy(x_hbm.at[wid], idx_v)

        def start_gather(j, b):
            pltpu.async_copy(w_hbm.at[idx_v.at[j]], rows_v.at[b], gsems[b])

        def start_out(j, b):
            pltpu.async_copy(
                rows_v.at[b].at[:, :dim], out_hbm.at[wid, j], osems[b]
            )

        def wait_out(j, b):
            pltpu.make_async_copy(
                rows_v.at[b].at[:, :dim], out_hbm.at[wid, j], osems[b]
            ).wait()

        for g in range(LEAD):
            start_gather(g, g % NBUF)

        def group(grp, carry):
            for b in range(NBUF):
                j = grp * NBUF + b
                pltpu.make_async_copy(
                    w_hbm.at[idx_v.at[j]], rows_v.at[b], gsems[b]
                ).wait()
                start_out(j, b)
                jn = j + LEAD
                bn = (b + LEAD) % NBUF

                @pl.when(jn < n_chunks)
                def _():
                    @pl.when(jn >= NBUF)
                    def _():
                        wait_out(jn - NBUF, bn)

                    start_gather(jn, bn)

            return carry

        lax.fori_loop(0, n_chunks // NBUF, group, 0, unroll=False)

        for b in range(NBUF):
            wait_out(n_chunks - NBUF + b, b)

    return emb_kernel


def kernel(x, weight):
    batch, hist = x.shape
    n_rows, dim = weight.shape
    total = batch * hist
    n_chunks = total // (NW * CHUNK)
    xf = x.reshape(NW, n_chunks, CHUNK).astype(jnp.int32)
    wpad = _build_detile(n_rows, dim)(weight)
    out = _build_gather(n_chunks, dim)(xf, wpad)
    return out.reshape(batch, hist, dim)


# speedup vs baseline: 2.3849x; 1.2204x over previous
import functools

import jax
import jax.numpy as jnp
from jax import lax
from jax.experimental import pallas as pl
from jax.experimental.pallas import tpu as pltpu
from jax.experimental.pallas import tpu_sc as plsc

NC = 2
NS = 16
NW = NC * NS
CHUNK = 128
PADDIM = 128
NBUF = 5
LEAD = 2


@functools.cache
def _build_gather(n_chunks: int, dim: int):
    assert n_chunks % NBUF == 0
    mesh = plsc.VectorSubcoreMesh(core_axis_name="c", subcore_axis_name="s")

    @functools.partial(
        pl.kernel,
        out_type=jax.ShapeDtypeStruct((NW, n_chunks, CHUNK, dim), jnp.float32),
        mesh=mesh,
        scratch_types=[
            pltpu.VMEM((n_chunks, CHUNK), jnp.int32),
            pltpu.VMEM((NBUF, CHUNK, PADDIM), jnp.float32),
            [pltpu.SemaphoreType.DMA] * NBUF,
            [pltpu.SemaphoreType.DMA] * NBUF,
        ],
        compiler_params=pltpu.CompilerParams(use_tc_tiling_on_sc=False),
    )
    def emb_kernel(x_hbm, w_hbm, out_hbm, idx_v, rows_v, gsems, osems):
        wid = lax.axis_index("s") * NC + lax.axis_index("c")
        pltpu.sync_copy(x_hbm.at[wid], idx_v)

        def start_gather(j, b):
            pltpu.async_copy(w_hbm.at[idx_v.at[j]], rows_v.at[b], gsems[b])

        def start_out(j, b):
            pltpu.async_copy(
                rows_v.at[b].at[:, :dim], out_hbm.at[wid, j], osems[b]
            )

        def wait_out(j, b):
            pltpu.make_async_copy(
                rows_v.at[b].at[:, :dim], out_hbm.at[wid, j], osems[b]
            ).wait()

        for g in range(LEAD):
            start_gather(g, g % NBUF)

        def group(grp, carry):
            for b in range(NBUF):
                j = grp * NBUF + b
                pltpu.make_async_copy(
                    w_hbm.at[idx_v.at[j]], rows_v.at[b], gsems[b]
                ).wait()
                start_out(j, b)
                jn = j + LEAD
                bn = (b + LEAD) % NBUF

                @pl.when(jn < n_chunks)
                def _():
                    @pl.when(jn >= NBUF)
                    def _():
                        wait_out(jn - NBUF, bn)

                    start_gather(jn, bn)

            return carry

        lax.fori_loop(0, n_chunks // NBUF, group, 0, unroll=False)

        for b in range(NBUF):
            wait_out(n_chunks - NBUF + b, b)

    return emb_kernel


def kernel(x, weight):
    batch, hist = x.shape
    n_rows, dim = weight.shape
    total = batch * hist
    n_chunks = total // (NW * CHUNK)
    xf = x.reshape(NW, n_chunks, CHUNK).astype(jnp.int32)
    wpad = jnp.pad(weight, ((0, 0), (0, PADDIM - dim)))
    out = _build_gather(n_chunks, dim)(xf, wpad)
    return out.reshape(batch, hist, dim)


# speedup vs baseline: 2.4055x; 1.0086x over previous
import functools

import jax
import jax.numpy as jnp
from jax import lax
from jax.experimental import pallas as pl
from jax.experimental.pallas import tpu as pltpu
from jax.experimental.pallas import tpu_sc as plsc

NC = 2
NS = 16
NW = NC * NS
CHUNK = 128
PADDIM = 128
NBUF = 5
LEAD = 3


@functools.cache
def _build_gather(n_chunks: int, dim: int):
    assert n_chunks % NBUF == 0
    mesh = plsc.VectorSubcoreMesh(core_axis_name="c", subcore_axis_name="s")

    @functools.partial(
        pl.kernel,
        out_type=jax.ShapeDtypeStruct((NW, n_chunks, CHUNK, dim), jnp.float32),
        mesh=mesh,
        scratch_types=[
            pltpu.VMEM((n_chunks, CHUNK), jnp.int32),
            pltpu.VMEM((NBUF, CHUNK, PADDIM), jnp.float32),
            [pltpu.SemaphoreType.DMA] * NBUF,
            [pltpu.SemaphoreType.DMA] * NBUF,
        ],
        compiler_params=pltpu.CompilerParams(use_tc_tiling_on_sc=False),
    )
    def emb_kernel(x_hbm, w_hbm, out_hbm, idx_v, rows_v, gsems, osems):
        wid = lax.axis_index("s") * NC + lax.axis_index("c")
        pltpu.sync_copy(x_hbm.at[wid], idx_v)

        def start_gather(j, b):
            pltpu.async_copy(w_hbm.at[idx_v.at[j]], rows_v.at[b], gsems[b])

        def start_out(j, b):
            pltpu.async_copy(
                rows_v.at[b].at[:, :dim], out_hbm.at[wid, j], osems[b]
            )

        def wait_out(j, b):
            pltpu.make_async_copy(
                rows_v.at[b].at[:, :dim], out_hbm.at[wid, j], osems[b]
            ).wait()

        for g in range(LEAD):
            start_gather(g, g % NBUF)

        def group(grp, carry):
            for b in range(NBUF):
                j = grp * NBUF + b
                pltpu.make_async_copy(
                    w_hbm.at[idx_v.at[j]], rows_v.at[b], gsems[b]
                ).wait()
                start_out(j, b)
                jn = j + LEAD
                bn = (b + LEAD) % NBUF

                @pl.when(jn < n_chunks)
                def _():
                    @pl.when(jn >= NBUF)
                    def _():
                        wait_out(jn - NBUF, bn)

                    start_gather(jn, bn)

            return carry

        lax.fori_loop(0, n_chunks // NBUF, group, 0, unroll=False)

        for b in range(NBUF):
            wait_out(n_chunks - NBUF + b, b)

    return emb_kernel


def kernel(x, weight):
    batch, hist = x.shape
    n_rows, dim = weight.shape
    total = batch * hist
    n_chunks = total // (NW * CHUNK)
    xf = x.reshape(NW, n_chunks, CHUNK).astype(jnp.int32)
    wpad = jnp.pad(weight, ((0, 0), (0, PADDIM - dim)))
    out = _build_gather(n_chunks, dim)(xf, wpad)
    return out.reshape(batch, hist, dim)
